# baseline (speedup 1.0000x reference)
import functools

import jax
import jax.numpy as jnp
from jax import lax
from jax.experimental import pallas as pl
from jax.experimental.pallas import tpu as pltpu
from jax.experimental.pallas import tpu_sc as plsc

DIM = 16
NN = 8
L = 16


def _splat(val):
    return jnp.full((L,), val, jnp.int32)


@functools.lru_cache(maxsize=None)
def _build(B, NC, NS):
    NW = NC * NS
    BW = B // NW
    CH = 128
    NCHUNK = BW // CH
    GPC = CH // L

    mesh = plsc.VectorSubcoreMesh(core_axis_name="c", subcore_axis_name="s")

    @functools.partial(
        pl.kernel,
        out_type=jax.ShapeDtypeStruct((B,), jnp.float32),
        mesh=mesh,
        compiler_params=pltpu.CompilerParams(
            needs_layout_passes=False, use_tc_tiling_on_sc=False),
        scratch_types=[
            pltpu.VMEM((BW,), jnp.int32),
            pltpu.VMEM((BW,), jnp.int32),
            pltpu.VMEM((64, DIM), jnp.float32),
            pltpu.VMEM((DIM, DIM), jnp.float32),
            pltpu.VMEM((DIM,), jnp.float32),
            pltpu.VMEM((CH, NN), jnp.int32),
            pltpu.VMEM((CH, NN), jnp.int32),
            pltpu.VMEM((CH, NN), jnp.int32),
            pltpu.VMEM((CH, NN), jnp.int32),
            pltpu.VMEM((CH, DIM), jnp.float32),
            pltpu.VMEM((CH, DIM), jnp.float32),
            pltpu.VMEM((CH * NN,), jnp.int32),
            pltpu.VMEM((CH * NN,), jnp.int32),
            pltpu.VMEM((CH * NN,), jnp.int32),
            pltpu.VMEM((CH * NN, DIM), jnp.float32),
            pltpu.VMEM((CH * NN, DIM), jnp.float32),
            pltpu.VMEM((CH * NN, DIM), jnp.float32),
            pltpu.VMEM((BW,), jnp.float32),
            pltpu.SemaphoreType.DMA,
            pltpu.SemaphoreType.DMA,
            pltpu.SemaphoreType.DMA,
            pltpu.SemaphoreType.DMA,
            pltpu.SemaphoreType.DMA,
            pltpu.SemaphoreType.DMA,
        ],
    )
    def klgcn(usr_h, ent_h, rel_h, w_h, b_h, u2i_h, i2u_h, ae_h, ar_h, u_h,
              v_h, out_h,
              u_v, v_v, rel_v, w_v, b_v, nb_u2i, nb_i2u, nb_ae, nb_ar,
              ue_r, io_r, fl_i2u, fl_u2i, fl_ae,
              usr_nb, ent_nb1, ent_nb2, out_v,
              s0, s1, s2, s3, s4, s5):
        wid = lax.axis_index("s") * NC + lax.axis_index("c")
        base = wid * BW
        pltpu.sync_copy(u_h.at[pl.ds(base, BW)], u_v)
        pltpu.sync_copy(v_h.at[pl.ds(base, BW)], v_v)
        pltpu.sync_copy(rel_h, rel_v)
        pltpu.sync_copy(w_h, w_v)
        pltpu.sync_copy(b_h, b_v)
        iota = lax.iota(jnp.int32, L)

        def chunk_body(c, carry):
            off = c * CH
            uc = u_v.at[pl.ds(off, CH)]
            vc = v_v.at[pl.ds(off, CH)]
            d_u2i = pltpu.async_copy(u2i_h.at[uc], nb_u2i, s0)
            d_i2u = pltpu.async_copy(i2u_h.at[vc], nb_i2u, s1)
            d_ae = pltpu.async_copy(ae_h.at[vc], nb_ae, s2)
            d_ar = pltpu.async_copy(ar_h.at[vc], nb_ar, s3)
            d_ue = pltpu.async_copy(usr_h.at[uc], ue_r, s4)
            d_io = pltpu.async_copy(ent_h.at[vc], io_r, s5)
            d_u2i.wait()
            d_i2u.wait()
            d_ae.wait()
            riota = jnp.right_shift(iota, 3)
            ciota = jnp.bitwise_and(iota, 7)

            def flat_body(i, cf):
                ridx = riota + i * 2
                o = i * L
                fl_i2u[pl.ds(o, L)] = plsc.load_gather(nb_i2u, [ridx, ciota])
                fl_u2i[pl.ds(o, L)] = plsc.load_gather(nb_u2i, [ridx, ciota])
                fl_ae[pl.ds(o, L)] = plsc.load_gather(nb_ae, [ridx, ciota])
                return cf

            lax.fori_loop(0, CH * NN // L, flat_body, 0)
            e_usr = pltpu.async_copy(usr_h.at[fl_i2u], usr_nb, s1)
            e_ent1 = pltpu.async_copy(ent_h.at[fl_u2i], ent_nb1, s0)
            e_ent2 = pltpu.async_copy(ent_h.at[fl_ae], ent_nb2, s2)
            d_ar.wait()
            d_ue.wait()
            d_io.wait()
            e_usr.wait()
            e_ent1.wait()
            e_ent2.wait()

            def group_body(g, carry2):
                rows = g * L + iota
                frows = [rows * NN + _splat(j) for j in range(NN)]
                ue = [plsc.load_gather(ue_r, [rows, _splat(d)])
                      for d in range(DIM)]
                s_list = []
                for j in range(NN):
                    relid = plsc.load_gather(nb_ar, [rows, _splat(j)])
                    acc = ue[0] * plsc.load_gather(rel_v, [relid, _splat(0)])
                    for d in range(1, DIM):
                        acc = acc + ue[d] * plsc.load_gather(
                            rel_v, [relid, _splat(d)])
                    s_list.append(acc)
                m = s_list[0]
                for j in range(1, NN):
                    m = jnp.maximum(m, s_list[j])
                e_list = [jnp.exp(sj - m) for sj in s_list]
                tot = e_list[0]
                for j in range(1, NN):
                    tot = tot + e_list[j]
                inv = 1.0 / tot
                p_list = [ej * inv for ej in e_list]
                y = [None] * DIM
                for d in range(DIM):
                    a = p_list[0] * plsc.load_gather(
                        ent_nb2, [frows[0], _splat(d)])
                    for j in range(1, NN):
                        a = a + p_list[j] * plsc.load_gather(
                            ent_nb2, [frows[j], _splat(d)])
                    x_d = a + plsc.load_gather(io_r, [rows, _splat(d)])
                    for dp in range(DIM):
                        w_sc = plsc.load_gather(w_v, [_splat(d), _splat(dp)])
                        term = x_d * w_sc
                        y[dp] = term if y[dp] is None else y[dp] + term
                t_list = []
                for dp in range(DIM):
                    yv = y[dp] + plsc.load_gather(b_v, [_splat(dp)])
                    tt = jnp.exp(jnp.abs(yv) * (-2.0))
                    r = (1.0 - tt) / (1.0 + tt)
                    t_list.append(jnp.where(yv < 0.0, -r, r))
                sc = None
                for d in range(DIM):
                    lu = plsc.load_gather(usr_nb, [frows[0], _splat(d)])
                    for j in range(1, NN):
                        lu = lu + plsc.load_gather(
                            usr_nb, [frows[j], _splat(d)])
                    li = plsc.load_gather(ent_nb1, [frows[0], _splat(d)])
                    for j in range(1, NN):
                        li = li + plsc.load_gather(
                            ent_nb1, [frows[j], _splat(d)])
                    uf = 0.0625 * lu + 0.5 * ue[d]
                    if_ = 0.0625 * li + 0.5 * t_list[d]
                    term = uf * if_
                    sc = term if sc is None else sc + term
                sig = 1.0 / (1.0 + jnp.exp(-sc))
                out_v[pl.ds(off + g * L, L)] = sig
                return carry2

            lax.fori_loop(0, GPC, group_body, 0)
            return carry

        lax.fori_loop(0, NCHUNK, chunk_body, 0)
        pltpu.sync_copy(out_v, out_h.at[pl.ds(base, BW)])

    return klgcn


def kernel(usr, ent, rel, W, b, u2i, i2u, adj_ent, adj_rel, u, v):
    B = u.shape[0]
    info = plsc.get_sparse_core_info()
    fn = _build(B, info.num_cores, info.num_subcores)
    return fn(usr.astype(jnp.float32), ent.astype(jnp.float32),
              rel.astype(jnp.float32), W.astype(jnp.float32),
              b.astype(jnp.float32), u2i.astype(jnp.int32),
              i2u.astype(jnp.int32), adj_ent.astype(jnp.int32),
              adj_rel.astype(jnp.int32), u.astype(jnp.int32),
              v.astype(jnp.int32))
